# Initial kernel scaffold; baseline (speedup 1.0000x reference)
import jax
import jax.numpy as jnp
from jax.experimental import pallas as pl


def kernel(x, table):
    raise NotImplementedError("write your pallas kernel here")


# speedup vs baseline: 1.8216x; 1.8216x over previous
import functools

import jax
import jax.numpy as jnp
from jax import lax
from jax.experimental import pallas as pl
from jax.experimental.pallas import tpu as pltpu
from jax.experimental.pallas import tpu_sc as plsc

MAXLEN = 8192
NROWS = 2 * MAXLEN - 1
D = 1024
SPLIT = MAXLEN - 1

NWORKERS = 32
ROWS_PER_W = 512
CHUNK = 32
NCHUNKS = ROWS_PER_W // CHUNK


def _copy_body(table, out, buf0, buf1, si0, si1, so0, so1):
    c = lax.axis_index("c")
    s = lax.axis_index("s")
    wid = s * 2 + c
    is_a = wid < 16
    dst0 = jnp.where(
        is_a,
        jnp.minimum(wid * ROWS_PER_W, SPLIT - ROWS_PER_W),
        SPLIT + (wid - 16) * ROWS_PER_W,
    )
    src0 = dst0 + jnp.where(is_a, MAXLEN, -SPLIT)
    dst0e = dst0 * D
    src0e = src0 * D
    ce = CHUNK * D

    bufs = (buf0, buf1)
    isems = (si0, si1)
    osems = (so0, so1)
    loads = [
        pltpu.make_async_copy(
            table.at[pl.ds(src0e + i * ce, ce)], bufs[i % 2], isems[i % 2]
        )
        for i in range(NCHUNKS)
    ]
    stores = [
        pltpu.make_async_copy(
            bufs[i % 2], out.at[pl.ds(dst0e + i * ce, ce)], osems[i % 2]
        )
        for i in range(NCHUNKS)
    ]

    loads[0].start()
    for i in range(NCHUNKS):
        if i + 1 < NCHUNKS:
            if i >= 1:
                stores[i - 1].wait()
            loads[i + 1].start()
        loads[i].wait()
        stores[i].start()
    stores[NCHUNKS - 2].wait()
    stores[NCHUNKS - 1].wait()


_shifted_copy = functools.partial(
    pl.kernel,
    mesh=plsc.VectorSubcoreMesh(core_axis_name="c", subcore_axis_name="s"),
    out_type=jax.ShapeDtypeStruct((NROWS * D,), jnp.float32),
    scratch_types=[
        pltpu.VMEM((CHUNK * D,), jnp.float32),
        pltpu.VMEM((CHUNK * D,), jnp.float32),
        pltpu.SemaphoreType.DMA,
        pltpu.SemaphoreType.DMA,
        pltpu.SemaphoreType.DMA,
        pltpu.SemaphoreType.DMA,
    ],
)(_copy_body)


def kernel(x, table):
    del x
    return _shifted_copy(table.reshape(-1)).reshape(NROWS, D)
